# baseline (speedup 1.0000x reference)
import functools

import jax
import jax.numpy as jnp
from jax import lax
from jax.experimental import pallas as pl
from jax.experimental.pallas import tpu as pltpu
from jax.experimental.pallas import tpu_sc as plsc

N = 4096
LANES = 16
BATCH = 8
NCHUNK = N // LANES


def _prep_body(dt_ref, ar, ai, wr, wi, th, lr0, li0, olr, oli, oamt):
    dtf = dt_ref[0, 0]
    er = jnp.exp(ar[...] * dtf)
    exp_r = er * jnp.cos(ai[...] * dtf)
    exp_i = er * jnp.sin(ai[...] * dtf)
    lr = lr0[...] * exp_r - li0[...] * exp_i
    li = lr0[...] * exp_i + li0[...] * exp_r
    olr[...] = lr
    oli[...] = li
    oamt[...] = wr[...] * lr + wi[...] * li - th[...]


def _prep(dtf, ar, ai, wr, wi, th, lr0, li0, interpret=False):
    shp = (N // 128, 128)
    vspec = pl.BlockSpec(memory_space=pltpu.VMEM)
    outs = pl.pallas_call(
        _prep_body,
        out_shape=[jax.ShapeDtypeStruct(shp, jnp.float32)] * 3,
        in_specs=[pl.BlockSpec(memory_space=pltpu.SMEM)] + [vspec] * 7,
        out_specs=[vspec] * 3,
        interpret=interpret,
    )(dtf.reshape(1, 1), ar.reshape(shp), ai.reshape(shp), wr.reshape(shp),
      wi.reshape(shp), th.reshape(shp), lr0.reshape(shp), li0.reshape(shp))
    return tuple(o.reshape(N) for o in outs)


def _build_collapse(interpret=False, num_cores=None, num_subcores=None):
    if num_cores is None:
        mesh = plsc.VectorSubcoreMesh(core_axis_name="c", subcore_axis_name="s")
    else:
        mesh = plsc.VectorSubcoreMesh(core_axis_name="c", subcore_axis_name="s",
                                      num_cores=num_cores,
                                      num_subcores=num_subcores)
    nc, ns = mesh.num_cores, mesh.num_subcores
    nw = nc * ns
    rpt = N // nw
    half = N // nc
    hchunks = half // LANES

    @functools.partial(
        pl.kernel,
        out_type=(jax.ShapeDtypeStruct((1, N), jnp.float32),
                  jax.ShapeDtypeStruct((1, N), jnp.float32)),
        mesh=mesh,
        interpret=interpret,
        compiler_params=pltpu.CompilerParams(needs_layout_passes=False),
        scratch_types=[
            pltpu.VMEM((N,), jnp.float32),
            pltpu.VMEM((N,), jnp.float32),
            pltpu.VMEM((half,), jnp.float32),
            pltpu.VMEM((half,), jnp.int32),
            pltpu.VMEM((LANES,), jnp.int32),
            pltpu.VMEM((LANES,), jnp.int32),
            pltpu.VMEM((BATCH, N), jnp.float32),
            pltpu.VMEM((BATCH, N), jnp.float32),
            pltpu.VMEM((rpt,), jnp.float32),
            pltpu.VMEM((rpt,), jnp.float32),
            pltpu.VMEM((LANES,), jnp.float32),
            pltpu.VMEM((LANES,), jnp.float32),
            pltpu.VMEM((half,), jnp.float32),
            pltpu.VMEM((half,), jnp.float32),
            pltpu.VMEM_SHARED((half,), jnp.int32),
            pltpu.VMEM_SHARED((LANES,), jnp.int32),
            pltpu.VMEM_SHARED((LANES,), jnp.int32),
            pltpu.VMEM_SHARED((half,), jnp.float32),
            pltpu.VMEM_SHARED((half,), jnp.float32),
            pltpu.SemaphoreType.DMA,
            pltpu.SemaphoreType.DMA,
        ],
    )
    def collapse(wr_hbm, wi_hbm, lr_hbm, li_hbm, amt_hbm,
                 outr_hbm, outi_hbm,
                 lr_v, li_v, amt_v, glist_v, bnd_v, tot_v, rowr_v, rowi_v,
                 outr_v, outi_v, fr8_v, fi8_v, resr_v, resi_v,
                 glist_s, bnd_s, tot_s, resr_s, resi_s, semr, semi):
        sc = lax.axis_index("c")
        tid = lax.axis_index("s")
        sc_base = sc * half
        row0 = sc_base + tid * rpt

        iota = lax.broadcasted_iota(jnp.int32, (LANES,), 0)
        lane0 = iota == 0

        pltpu.async_copy(lr_hbm, lr_v, semr)
        pltpu.async_copy(li_hbm, li_v, semr)
        pltpu.async_copy(lr_hbm.at[pl.ds(row0, rpt)], outr_v, semi)
        pltpu.async_copy(li_hbm.at[pl.ds(row0, rpt)], outi_v, semi)

        @pl.when(tid == 0)
        def _():
            pltpu.sync_copy(amt_hbm.at[pl.ds(sc_base, half)], amt_v)
            base_v = jnp.full((LANES,), sc_base, jnp.int32)
            for k in range(hchunks):
                glist_v[pl.ds(k * LANES, LANES)] = base_v
            cnt = jnp.int32(0)
            for k in range(hchunks):
                if k % (rpt // LANES) == 0:
                    plsc.store_scatter(
                        bnd_v, [jnp.full((LANES,), k // (rpt // LANES),
                                         jnp.int32)],
                        jnp.full((LANES,), cnt, jnp.int32), mask=lane0)
                m = amt_v[pl.ds(k * LANES, LANES)] >= 0.0
                mi = m.astype(jnp.int32)
                pos = cnt + jnp.cumsum(mi) - 1
                plsc.store_scatter(glist_v, [pos],
                                   base_v + (k * LANES) + iota, mask=m)
                cnt = cnt + jnp.sum(mi)
            tot_v[...] = jnp.full((LANES,), cnt, jnp.int32)
            pltpu.sync_copy(glist_v, glist_s)
            pltpu.sync_copy(bnd_v, bnd_s)
            pltpu.sync_copy(tot_v, tot_s)

        plsc.subcore_barrier()

        pltpu.sync_copy(glist_s, glist_v)
        pltpu.sync_copy(bnd_s, bnd_v)
        pltpu.sync_copy(tot_s, tot_v)
        total = tot_v[pl.ds(0, LANES)][0]

        share = ((total + (8 * ns - 1)) // (8 * ns)) * 8
        start = tid * share
        myn = jnp.maximum(0, jnp.minimum(total - start, share))
        nb = (myn + (BATCH - 1)) // BATCH
        zeros = jnp.zeros((LANES,), jnp.float32)

        pltpu.make_async_copy(lr_hbm.at[pl.ds(row0, rpt)], outr_v, semi).wait()
        pltpu.make_async_copy(li_hbm.at[pl.ds(row0, rpt)], outi_v, semi).wait()
        pltpu.make_async_copy(lr_hbm, lr_v, semr).wait()
        pltpu.make_async_copy(li_hbm, li_v, semr).wait()

        @pl.when(nb > 0)
        def _():
            pltpu.async_copy(wr_hbm.at[glist_v.at[pl.ds(start, BATCH)]],
                             rowr_v, semr)

        def batch_body(b, carry):
            sl = glist_v.at[pl.ds(start + b * BATCH, BATCH)]
            pltpu.make_async_copy(wr_hbm.at[sl], rowr_v, semr).wait()
            pltpu.async_copy(wi_hbm.at[sl], rowi_v, semi)

            init = (tuple([zeros] * BATCH), tuple([zeros] * BATCH))

            @plsc.parallel_loop(0, NCHUNK, unroll=8, carry=init)
            def acc_a(c, accs):
                fr_t, fi_t = accs
                base = c * LANES
                lr_c = lr_v[pl.ds(base, LANES)]
                li_c = li_v[pl.ds(base, LANES)]
                nfr, nfi = [], []
                for r in range(BATCH):
                    w_r = rowr_v[r, pl.ds(base, LANES)]
                    nfr.append(fr_t[r] + w_r * lr_c)
                    nfi.append(fi_t[r] + w_r * li_c)
                return (tuple(nfr), tuple(nfi))

            fr_t, fi_t = acc_a

            pltpu.make_async_copy(wi_hbm.at[sl], rowi_v, semi).wait()

            @pl.when(b + 1 < nb)
            def _():
                pltpu.async_copy(
                    wr_hbm.at[glist_v.at[pl.ds(start + (b + 1) * BATCH,
                                               BATCH)]],
                    rowr_v, semr)

            @plsc.parallel_loop(0, NCHUNK, unroll=8, carry=(fr_t, fi_t))
            def acc_b(c, accs):
                fr_t, fi_t = accs
                base = c * LANES
                lr_c = lr_v[pl.ds(base, LANES)]
                li_c = li_v[pl.ds(base, LANES)]
                nfr, nfi = [], []
                for r in range(BATCH):
                    w_i = rowi_v[r, pl.ds(base, LANES)]
                    nfr.append(fr_t[r] - w_i * li_c)
                    nfi.append(fi_t[r] + w_i * lr_c)
                return (tuple(nfr), tuple(nfi))

            fr_t, fi_t = acc_b

            for r in range(BATCH):
                rl = jnp.full((LANES,), r, jnp.int32)
                plsc.store_scatter(fr8_v, [rl],
                                   jnp.full((LANES,), jnp.sum(fr_t[r])),
                                   mask=lane0)
                plsc.store_scatter(fi8_v, [rl],
                                   jnp.full((LANES,), jnp.sum(fi_t[r])),
                                   mask=lane0)
            pltpu.sync_copy(fr8_v.at[pl.ds(0, BATCH)],
                            resr_s.at[pl.ds(start + b * BATCH, BATCH)])
            pltpu.sync_copy(fi8_v.at[pl.ds(0, BATCH)],
                            resi_s.at[pl.ds(start + b * BATCH, BATCH)])
            return carry

        lax.fori_loop(0, nb, batch_body, jnp.int32(0))

        plsc.subcore_barrier()

        pltpu.sync_copy(resr_s, resr_v)
        pltpu.sync_copy(resi_s, resi_v)
        tid_v = jnp.full((LANES,), tid, jnp.int32)
        lo = plsc.load_gather(bnd_v, [tid_v])[0]
        hi_next = plsc.load_gather(bnd_v, [jnp.minimum(tid_v + 1, ns - 1)])[0]
        hi = jnp.where(tid == ns - 1, total, hi_next)
        row0_v = jnp.full((LANES,), row0, jnp.int32)

        def owner_body(s, carry):
            s_v = jnp.full((LANES,), s, jnp.int32)
            g_v = plsc.load_gather(glist_v, [s_v])
            l_v = g_v - row0_v
            frv = plsc.load_gather(resr_v, [s_v])
            fiv = plsc.load_gather(resi_v, [s_v])
            lrv = plsc.load_gather(lr_v, [g_v])
            liv = plsc.load_gather(li_v, [g_v])
            plsc.store_scatter(outr_v, [l_v],
                               lrv * jnp.maximum(frv, 0.0), mask=lane0)
            plsc.store_scatter(outi_v, [l_v], liv * fiv, mask=lane0)
            return carry

        lax.fori_loop(lo, hi, owner_body, jnp.int32(0))

        pltpu.sync_copy(outr_v, outr_hbm.at[0, pl.ds(row0, rpt)])
        pltpu.sync_copy(outi_v, outi_hbm.at[0, pl.ds(row0, rpt)])

    return collapse


def kernel(t_span, dt, A_real, A_imag, w_acc_real, w_acc_imag, theta,
           W_filter_real, W_filter_imag, L_real_init, L_imag_init):
    num_steps = t_span.shape[0] - 1
    dtf = jnp.asarray(dt, jnp.float32)
    collapse = _build_collapse()
    Lr, Li = L_real_init, L_imag_init
    reals, imags = [], []
    for _ in range(num_steps):
        lr1, li1, amt = _prep(dtf, A_real, A_imag, w_acc_real, w_acc_imag,
                              theta, Lr, Li)
        o_r, o_i = collapse(W_filter_real, W_filter_imag, lr1, li1, amt)
        Lr, Li = o_r[0], o_i[0]
        reals.append(o_r)
        imags.append(o_i)
    if num_steps == 1:
        return reals[0], imags[0]
    return jnp.concatenate(reals), jnp.concatenate(imags)


# speedup vs baseline: 1.2685x; 1.2685x over previous
import functools

import jax
import jax.numpy as jnp
from jax import lax
from jax.experimental import pallas as pl
from jax.experimental.pallas import tpu as pltpu
from jax.experimental.pallas import tpu_sc as plsc

N = 4096
LANES = 16
BATCH = 8
NCHUNK = N // LANES


def _prep_body(dt_ref, ar, ai, wr, wi, th, lr0, li0, olr, oli, oamt):
    dtf = dt_ref[0, 0]
    er = jnp.exp(ar[...] * dtf)
    exp_r = er * jnp.cos(ai[...] * dtf)
    exp_i = er * jnp.sin(ai[...] * dtf)
    lr = lr0[...] * exp_r - li0[...] * exp_i
    li = lr0[...] * exp_i + li0[...] * exp_r
    olr[...] = lr
    oli[...] = li
    oamt[...] = wr[...] * lr + wi[...] * li - th[...]


def _prep(dtf, ar, ai, wr, wi, th, lr0, li0, interpret=False):
    shp = (N // 128, 128)
    vspec = pl.BlockSpec(memory_space=pltpu.VMEM)
    outs = pl.pallas_call(
        _prep_body,
        out_shape=[jax.ShapeDtypeStruct(shp, jnp.float32)] * 3,
        in_specs=[pl.BlockSpec(memory_space=pltpu.SMEM)] + [vspec] * 7,
        out_specs=[vspec] * 3,
        interpret=interpret,
    )(dtf.reshape(1, 1), ar.reshape(shp), ai.reshape(shp), wr.reshape(shp),
      wi.reshape(shp), th.reshape(shp), lr0.reshape(shp), li0.reshape(shp))
    return tuple(o.reshape(N) for o in outs)


def _build_collapse(interpret=False, num_cores=None, num_subcores=None):
    if num_cores is None:
        mesh = plsc.VectorSubcoreMesh(core_axis_name="c", subcore_axis_name="s")
    else:
        mesh = plsc.VectorSubcoreMesh(core_axis_name="c", subcore_axis_name="s",
                                      num_cores=num_cores,
                                      num_subcores=num_subcores)
    nc, ns = mesh.num_cores, mesh.num_subcores
    nw = nc * ns
    rpt = N // nw
    half = N // nc
    hchunks = half // LANES

    @functools.partial(
        pl.kernel,
        out_type=(jax.ShapeDtypeStruct((1, N), jnp.float32),
                  jax.ShapeDtypeStruct((1, N), jnp.float32)),
        mesh=mesh,
        interpret=interpret,
        compiler_params=pltpu.CompilerParams(needs_layout_passes=False),
        scratch_types=[
            pltpu.VMEM((N,), jnp.float32),
            pltpu.VMEM((N,), jnp.float32),
            pltpu.VMEM((half,), jnp.float32),
            pltpu.VMEM((half,), jnp.int32),
            pltpu.VMEM((LANES,), jnp.int32),
            pltpu.VMEM((LANES,), jnp.int32),
            pltpu.VMEM((BATCH, N), jnp.float32),
            pltpu.VMEM((BATCH, N), jnp.float32),
            pltpu.VMEM((rpt,), jnp.float32),
            pltpu.VMEM((rpt,), jnp.float32),
            pltpu.VMEM((LANES,), jnp.float32),
            pltpu.VMEM((LANES,), jnp.float32),
            pltpu.VMEM((half,), jnp.float32),
            pltpu.VMEM((half,), jnp.float32),
            pltpu.VMEM_SHARED((half,), jnp.int32),
            pltpu.VMEM_SHARED((LANES,), jnp.int32),
            pltpu.VMEM_SHARED((LANES,), jnp.int32),
            pltpu.VMEM_SHARED((half,), jnp.float32),
            pltpu.VMEM_SHARED((half,), jnp.float32),
            pltpu.SemaphoreType.DMA,
            pltpu.SemaphoreType.DMA,
        ],
    )
    def collapse(wr_hbm, wi_hbm, lr_hbm, li_hbm, amt_hbm,
                 outr_hbm, outi_hbm,
                 lr_v, li_v, amt_v, glist_v, bnd_v, tot_v, rowr_v, rowi_v,
                 outr_v, outi_v, fr8_v, fi8_v, resr_v, resi_v,
                 glist_s, bnd_s, tot_s, resr_s, resi_s, semr, semi):
        sc = lax.axis_index("c")
        tid = lax.axis_index("s")
        sc_base = sc * half
        row0 = sc_base + tid * rpt

        iota = lax.broadcasted_iota(jnp.int32, (LANES,), 0)
        lane0 = iota == 0

        pltpu.async_copy(lr_hbm, lr_v, semr)
        pltpu.async_copy(li_hbm, li_v, semr)
        pltpu.async_copy(lr_hbm.at[pl.ds(row0, rpt)], outr_v, semi)
        pltpu.async_copy(li_hbm.at[pl.ds(row0, rpt)], outi_v, semi)

        @pl.when(tid == 0)
        def _():
            pltpu.sync_copy(amt_hbm.at[pl.ds(sc_base, half)], amt_v)
            base_v = jnp.full((LANES,), sc_base, jnp.int32)
            for k in range(hchunks):
                glist_v[pl.ds(k * LANES, LANES)] = base_v
            cnt = jnp.int32(0)
            for k in range(hchunks):
                if k % (rpt // LANES) == 0:
                    plsc.store_scatter(
                        bnd_v, [jnp.full((LANES,), k // (rpt // LANES),
                                         jnp.int32)],
                        jnp.full((LANES,), cnt, jnp.int32), mask=lane0)
                m = amt_v[pl.ds(k * LANES, LANES)] >= 0.0
                mi = m.astype(jnp.int32)
                pos = cnt + jnp.cumsum(mi) - 1
                plsc.store_scatter(glist_v, [pos],
                                   base_v + (k * LANES) + iota, mask=m)
                cnt = cnt + jnp.sum(mi)
            tot_v[...] = jnp.full((LANES,), cnt, jnp.int32)
            pltpu.sync_copy(glist_v, glist_s)
            pltpu.sync_copy(bnd_v, bnd_s)
            pltpu.sync_copy(tot_v, tot_s)

        plsc.subcore_barrier()

        pltpu.sync_copy(glist_s, glist_v)
        pltpu.sync_copy(bnd_s, bnd_v)
        pltpu.sync_copy(tot_s, tot_v)
        total = tot_v[pl.ds(0, LANES)][0]

        share = ((total + (8 * ns - 1)) // (8 * ns)) * 8
        start = tid * share
        myn = jnp.maximum(0, jnp.minimum(total - start, share))
        nb = (myn + (BATCH - 1)) // BATCH
        zeros = jnp.zeros((LANES,), jnp.float32)

        pltpu.make_async_copy(lr_hbm.at[pl.ds(row0, rpt)], outr_v, semi).wait()
        pltpu.make_async_copy(li_hbm.at[pl.ds(row0, rpt)], outi_v, semi).wait()
        pltpu.make_async_copy(lr_hbm, lr_v, semr).wait()
        pltpu.make_async_copy(li_hbm, li_v, semr).wait()

        @pl.when(nb > 0)
        def _():
            pltpu.async_copy(wr_hbm.at[glist_v.at[pl.ds(start, BATCH)]],
                             rowr_v, semr)

        def batch_body(b, carry):
            sl = glist_v.at[pl.ds(start + b * BATCH, BATCH)]
            pltpu.make_async_copy(wr_hbm.at[sl], rowr_v, semr).wait()
            pltpu.async_copy(wi_hbm.at[sl], rowi_v, semi)

            init = (tuple([zeros] * BATCH), tuple([zeros] * BATCH))

            @plsc.parallel_loop(0, NCHUNK, unroll=2, carry=init)
            def acc_a(c, accs):
                fr_t, fi_t = accs
                base = c * LANES
                lr_c = lr_v[pl.ds(base, LANES)]
                li_c = li_v[pl.ds(base, LANES)]
                nfr, nfi = [], []
                for r in range(BATCH):
                    w_r = rowr_v[r, pl.ds(base, LANES)]
                    nfr.append(fr_t[r] + w_r * lr_c)
                    nfi.append(fi_t[r] + w_r * li_c)
                return (tuple(nfr), tuple(nfi))

            fr_t, fi_t = acc_a

            pltpu.make_async_copy(wi_hbm.at[sl], rowi_v, semi).wait()

            @pl.when(b + 1 < nb)
            def _():
                pltpu.async_copy(
                    wr_hbm.at[glist_v.at[pl.ds(start + (b + 1) * BATCH,
                                               BATCH)]],
                    rowr_v, semr)

            @plsc.parallel_loop(0, NCHUNK, unroll=2, carry=(fr_t, fi_t))
            def acc_b(c, accs):
                fr_t, fi_t = accs
                base = c * LANES
                lr_c = lr_v[pl.ds(base, LANES)]
                li_c = li_v[pl.ds(base, LANES)]
                nfr, nfi = [], []
                for r in range(BATCH):
                    w_i = rowi_v[r, pl.ds(base, LANES)]
                    nfr.append(fr_t[r] - w_i * li_c)
                    nfi.append(fi_t[r] + w_i * lr_c)
                return (tuple(nfr), tuple(nfi))

            fr_t, fi_t = acc_b

            for r in range(BATCH):
                rl = jnp.full((LANES,), r, jnp.int32)
                plsc.store_scatter(fr8_v, [rl],
                                   jnp.full((LANES,), jnp.sum(fr_t[r])),
                                   mask=lane0)
                plsc.store_scatter(fi8_v, [rl],
                                   jnp.full((LANES,), jnp.sum(fi_t[r])),
                                   mask=lane0)
            pltpu.sync_copy(fr8_v.at[pl.ds(0, BATCH)],
                            resr_s.at[pl.ds(start + b * BATCH, BATCH)])
            pltpu.sync_copy(fi8_v.at[pl.ds(0, BATCH)],
                            resi_s.at[pl.ds(start + b * BATCH, BATCH)])
            return carry

        lax.fori_loop(0, nb, batch_body, jnp.int32(0))

        plsc.subcore_barrier()

        pltpu.sync_copy(resr_s, resr_v)
        pltpu.sync_copy(resi_s, resi_v)
        tid_v = jnp.full((LANES,), tid, jnp.int32)
        lo = plsc.load_gather(bnd_v, [tid_v])[0]
        hi_next = plsc.load_gather(bnd_v, [jnp.minimum(tid_v + 1, ns - 1)])[0]
        hi = jnp.where(tid == ns - 1, total, hi_next)
        row0_v = jnp.full((LANES,), row0, jnp.int32)

        def owner_body(s, carry):
            s_v = jnp.full((LANES,), s, jnp.int32)
            g_v = plsc.load_gather(glist_v, [s_v])
            l_v = g_v - row0_v
            frv = plsc.load_gather(resr_v, [s_v])
            fiv = plsc.load_gather(resi_v, [s_v])
            lrv = plsc.load_gather(lr_v, [g_v])
            liv = plsc.load_gather(li_v, [g_v])
            plsc.store_scatter(outr_v, [l_v],
                               lrv * jnp.maximum(frv, 0.0), mask=lane0)
            plsc.store_scatter(outi_v, [l_v], liv * fiv, mask=lane0)
            return carry

        lax.fori_loop(lo, hi, owner_body, jnp.int32(0))

        pltpu.sync_copy(outr_v, outr_hbm.at[0, pl.ds(row0, rpt)])
        pltpu.sync_copy(outi_v, outi_hbm.at[0, pl.ds(row0, rpt)])

    return collapse


def kernel(t_span, dt, A_real, A_imag, w_acc_real, w_acc_imag, theta,
           W_filter_real, W_filter_imag, L_real_init, L_imag_init):
    num_steps = t_span.shape[0] - 1
    dtf = jnp.asarray(dt, jnp.float32)
    collapse = _build_collapse()
    Lr, Li = L_real_init, L_imag_init
    reals, imags = [], []
    for _ in range(num_steps):
        lr1, li1, amt = _prep(dtf, A_real, A_imag, w_acc_real, w_acc_imag,
                              theta, Lr, Li)
        o_r, o_i = collapse(W_filter_real, W_filter_imag, lr1, li1, amt)
        Lr, Li = o_r[0], o_i[0]
        reals.append(o_r)
        imags.append(o_i)
    if num_steps == 1:
        return reals[0], imags[0]
    return jnp.concatenate(reals), jnp.concatenate(imags)


# speedup vs baseline: 1.2732x; 1.0037x over previous
import functools

import jax
import jax.numpy as jnp
from jax import lax
from jax.experimental import pallas as pl
from jax.experimental.pallas import tpu as pltpu
from jax.experimental.pallas import tpu_sc as plsc

N = 4096
LANES = 16
BATCH = 8
NCHUNK = N // LANES


def _prep_body(dt_ref, ar, ai, wr, wi, th, lr0, li0, olr, oli, oamt):
    dtf = dt_ref[0, 0]
    er = jnp.exp(ar[...] * dtf)
    exp_r = er * jnp.cos(ai[...] * dtf)
    exp_i = er * jnp.sin(ai[...] * dtf)
    lr = lr0[...] * exp_r - li0[...] * exp_i
    li = lr0[...] * exp_i + li0[...] * exp_r
    olr[...] = lr
    oli[...] = li
    oamt[...] = wr[...] * lr + wi[...] * li - th[...]


def _prep(dtf, ar, ai, wr, wi, th, lr0, li0, interpret=False):
    shp = (N // 128, 128)
    vspec = pl.BlockSpec(memory_space=pltpu.VMEM)
    outs = pl.pallas_call(
        _prep_body,
        out_shape=[jax.ShapeDtypeStruct(shp, jnp.float32)] * 3,
        in_specs=[pl.BlockSpec(memory_space=pltpu.SMEM)] + [vspec] * 7,
        out_specs=[vspec] * 3,
        interpret=interpret,
    )(dtf.reshape(1, 1), ar.reshape(shp), ai.reshape(shp), wr.reshape(shp),
      wi.reshape(shp), th.reshape(shp), lr0.reshape(shp), li0.reshape(shp))
    return tuple(o.reshape(N) for o in outs)


def _build_collapse(interpret=False, num_cores=None, num_subcores=None):
    if num_cores is None:
        mesh = plsc.VectorSubcoreMesh(core_axis_name="c", subcore_axis_name="s")
    else:
        mesh = plsc.VectorSubcoreMesh(core_axis_name="c", subcore_axis_name="s",
                                      num_cores=num_cores,
                                      num_subcores=num_subcores)
    nc, ns = mesh.num_cores, mesh.num_subcores
    nw = nc * ns
    rpt = N // nw
    half = N // nc
    hchunks = half // LANES

    @functools.partial(
        pl.kernel,
        out_type=(jax.ShapeDtypeStruct((1, N), jnp.float32),
                  jax.ShapeDtypeStruct((1, N), jnp.float32)),
        mesh=mesh,
        interpret=interpret,
        compiler_params=pltpu.CompilerParams(needs_layout_passes=False),
        scratch_types=[
            pltpu.VMEM((N,), jnp.float32),
            pltpu.VMEM((N,), jnp.float32),
            pltpu.VMEM((half,), jnp.float32),
            pltpu.VMEM((half,), jnp.int32),
            pltpu.VMEM((LANES,), jnp.int32),
            pltpu.VMEM((LANES,), jnp.int32),
            pltpu.VMEM((BATCH, N), jnp.float32),
            pltpu.VMEM((BATCH, N), jnp.float32),
            pltpu.VMEM((rpt,), jnp.float32),
            pltpu.VMEM((rpt,), jnp.float32),
            pltpu.VMEM((LANES,), jnp.float32),
            pltpu.VMEM((LANES,), jnp.float32),
            pltpu.VMEM((half,), jnp.float32),
            pltpu.VMEM((half,), jnp.float32),
            pltpu.VMEM_SHARED((half,), jnp.int32),
            pltpu.VMEM_SHARED((LANES,), jnp.int32),
            pltpu.VMEM_SHARED((LANES,), jnp.int32),
            pltpu.VMEM_SHARED((half,), jnp.float32),
            pltpu.VMEM_SHARED((half,), jnp.float32),
            pltpu.SemaphoreType.DMA,
            pltpu.SemaphoreType.DMA,
        ],
    )
    def collapse(wr_hbm, wi_hbm, lr_hbm, li_hbm, amt_hbm,
                 outr_hbm, outi_hbm,
                 lr_v, li_v, amt_v, glist_v, bnd_v, tot_v, rowr_v, rowi_v,
                 outr_v, outi_v, fr8_v, fi8_v, resr_v, resi_v,
                 glist_s, bnd_s, tot_s, resr_s, resi_s, semr, semi):
        sc = lax.axis_index("c")
        tid = lax.axis_index("s")
        sc_base = sc * half
        row0 = sc_base + tid * rpt

        iota = lax.broadcasted_iota(jnp.int32, (LANES,), 0)
        lane0 = iota == 0

        pltpu.async_copy(lr_hbm, lr_v, semr)
        pltpu.async_copy(li_hbm, li_v, semr)
        pltpu.async_copy(lr_hbm.at[pl.ds(row0, rpt)], outr_v, semi)
        pltpu.async_copy(li_hbm.at[pl.ds(row0, rpt)], outi_v, semi)

        @pl.when(tid == 0)
        def _():
            pltpu.sync_copy(amt_hbm.at[pl.ds(sc_base, half)], amt_v)
            base_v = jnp.full((LANES,), sc_base, jnp.int32)
            for k in range(hchunks):
                glist_v[pl.ds(k * LANES, LANES)] = base_v
            cnt = jnp.int32(0)
            for k in range(hchunks):
                if k % (rpt // LANES) == 0:
                    plsc.store_scatter(
                        bnd_v, [jnp.full((LANES,), k // (rpt // LANES),
                                         jnp.int32)],
                        jnp.full((LANES,), cnt, jnp.int32), mask=lane0)
                m = amt_v[pl.ds(k * LANES, LANES)] >= 0.0
                mi = m.astype(jnp.int32)
                pos = cnt + jnp.cumsum(mi) - 1
                plsc.store_scatter(glist_v, [pos],
                                   base_v + (k * LANES) + iota, mask=m)
                cnt = cnt + jnp.sum(mi)
            tot_v[...] = jnp.full((LANES,), cnt, jnp.int32)
            pltpu.sync_copy(glist_v, glist_s)
            pltpu.sync_copy(bnd_v, bnd_s)
            pltpu.sync_copy(tot_v, tot_s)

        plsc.subcore_barrier()

        pltpu.sync_copy(glist_s, glist_v)
        pltpu.sync_copy(bnd_s, bnd_v)
        pltpu.sync_copy(tot_s, tot_v)
        total = tot_v[pl.ds(0, LANES)][0]

        share = ((total + (8 * ns - 1)) // (8 * ns)) * 8
        start = tid * share
        myn = jnp.maximum(0, jnp.minimum(total - start, share))
        nb = (myn + (BATCH - 1)) // BATCH
        zeros = jnp.zeros((LANES,), jnp.float32)

        pltpu.make_async_copy(lr_hbm.at[pl.ds(row0, rpt)], outr_v, semi).wait()
        pltpu.make_async_copy(li_hbm.at[pl.ds(row0, rpt)], outi_v, semi).wait()
        pltpu.make_async_copy(lr_hbm, lr_v, semr).wait()
        pltpu.make_async_copy(li_hbm, li_v, semr).wait()

        @pl.when(nb > 0)
        def _():
            pltpu.async_copy(wr_hbm.at[glist_v.at[pl.ds(start, BATCH)]],
                             rowr_v, semr)

        def batch_body(b, carry):
            sl = glist_v.at[pl.ds(start + b * BATCH, BATCH)]
            pltpu.make_async_copy(wr_hbm.at[sl], rowr_v, semr).wait()
            pltpu.async_copy(wi_hbm.at[sl], rowi_v, semi)

            init = (tuple([zeros] * BATCH), tuple([zeros] * BATCH))

            @plsc.parallel_loop(0, NCHUNK, unroll=1, carry=init)
            def acc_a(c, accs):
                fr_t, fi_t = accs
                base = c * LANES
                lr_c = lr_v[pl.ds(base, LANES)]
                li_c = li_v[pl.ds(base, LANES)]
                nfr, nfi = [], []
                for r in range(BATCH):
                    w_r = rowr_v[r, pl.ds(base, LANES)]
                    nfr.append(fr_t[r] + w_r * lr_c)
                    nfi.append(fi_t[r] + w_r * li_c)
                return (tuple(nfr), tuple(nfi))

            fr_t, fi_t = acc_a

            pltpu.make_async_copy(wi_hbm.at[sl], rowi_v, semi).wait()

            @pl.when(b + 1 < nb)
            def _():
                pltpu.async_copy(
                    wr_hbm.at[glist_v.at[pl.ds(start + (b + 1) * BATCH,
                                               BATCH)]],
                    rowr_v, semr)

            @plsc.parallel_loop(0, NCHUNK, unroll=1, carry=(fr_t, fi_t))
            def acc_b(c, accs):
                fr_t, fi_t = accs
                base = c * LANES
                lr_c = lr_v[pl.ds(base, LANES)]
                li_c = li_v[pl.ds(base, LANES)]
                nfr, nfi = [], []
                for r in range(BATCH):
                    w_i = rowi_v[r, pl.ds(base, LANES)]
                    nfr.append(fr_t[r] - w_i * li_c)
                    nfi.append(fi_t[r] + w_i * lr_c)
                return (tuple(nfr), tuple(nfi))

            fr_t, fi_t = acc_b

            for r in range(BATCH):
                rl = jnp.full((LANES,), r, jnp.int32)
                plsc.store_scatter(fr8_v, [rl],
                                   jnp.full((LANES,), jnp.sum(fr_t[r])),
                                   mask=lane0)
                plsc.store_scatter(fi8_v, [rl],
                                   jnp.full((LANES,), jnp.sum(fi_t[r])),
                                   mask=lane0)
            pltpu.sync_copy(fr8_v.at[pl.ds(0, BATCH)],
                            resr_s.at[pl.ds(start + b * BATCH, BATCH)])
            pltpu.sync_copy(fi8_v.at[pl.ds(0, BATCH)],
                            resi_s.at[pl.ds(start + b * BATCH, BATCH)])
            return carry

        lax.fori_loop(0, nb, batch_body, jnp.int32(0))

        plsc.subcore_barrier()

        pltpu.sync_copy(resr_s, resr_v)
        pltpu.sync_copy(resi_s, resi_v)
        tid_v = jnp.full((LANES,), tid, jnp.int32)
        lo = plsc.load_gather(bnd_v, [tid_v])[0]
        hi_next = plsc.load_gather(bnd_v, [jnp.minimum(tid_v + 1, ns - 1)])[0]
        hi = jnp.where(tid == ns - 1, total, hi_next)
        row0_v = jnp.full((LANES,), row0, jnp.int32)

        def owner_body(s, carry):
            s_v = jnp.full((LANES,), s, jnp.int32)
            g_v = plsc.load_gather(glist_v, [s_v])
            l_v = g_v - row0_v
            frv = plsc.load_gather(resr_v, [s_v])
            fiv = plsc.load_gather(resi_v, [s_v])
            lrv = plsc.load_gather(lr_v, [g_v])
            liv = plsc.load_gather(li_v, [g_v])
            plsc.store_scatter(outr_v, [l_v],
                               lrv * jnp.maximum(frv, 0.0), mask=lane0)
            plsc.store_scatter(outi_v, [l_v], liv * fiv, mask=lane0)
            return carry

        lax.fori_loop(lo, hi, owner_body, jnp.int32(0))

        pltpu.sync_copy(outr_v, outr_hbm.at[0, pl.ds(row0, rpt)])
        pltpu.sync_copy(outi_v, outi_hbm.at[0, pl.ds(row0, rpt)])

    return collapse


def kernel(t_span, dt, A_real, A_imag, w_acc_real, w_acc_imag, theta,
           W_filter_real, W_filter_imag, L_real_init, L_imag_init):
    num_steps = t_span.shape[0] - 1
    dtf = jnp.asarray(dt, jnp.float32)
    collapse = _build_collapse()
    Lr, Li = L_real_init, L_imag_init
    reals, imags = [], []
    for _ in range(num_steps):
        lr1, li1, amt = _prep(dtf, A_real, A_imag, w_acc_real, w_acc_imag,
                              theta, Lr, Li)
        o_r, o_i = collapse(W_filter_real, W_filter_imag, lr1, li1, amt)
        Lr, Li = o_r[0], o_i[0]
        reals.append(o_r)
        imags.append(o_i)
    if num_steps == 1:
        return reals[0], imags[0]
    return jnp.concatenate(reals), jnp.concatenate(imags)


# speedup vs baseline: 1.2745x; 1.0010x over previous
import functools

import jax
import jax.numpy as jnp
from jax import lax
from jax.experimental import pallas as pl
from jax.experimental.pallas import tpu as pltpu
from jax.experimental.pallas import tpu_sc as plsc

N = 4096
LANES = 16
BATCH = 8
NCHUNK = N // LANES


def _prep_body(dt_ref, ar, ai, wr, wi, th, lr0, li0, olr, oli, oamt):
    dtf = dt_ref[0, 0]
    er = jnp.exp(ar[...] * dtf)
    exp_r = er * jnp.cos(ai[...] * dtf)
    exp_i = er * jnp.sin(ai[...] * dtf)
    lr = lr0[...] * exp_r - li0[...] * exp_i
    li = lr0[...] * exp_i + li0[...] * exp_r
    olr[...] = lr
    oli[...] = li
    oamt[...] = wr[...] * lr + wi[...] * li - th[...]


def _prep(dtf, ar, ai, wr, wi, th, lr0, li0, interpret=False):
    shp = (N // 128, 128)
    vspec = pl.BlockSpec(memory_space=pltpu.VMEM)
    outs = pl.pallas_call(
        _prep_body,
        out_shape=[jax.ShapeDtypeStruct(shp, jnp.float32)] * 3,
        in_specs=[pl.BlockSpec(memory_space=pltpu.SMEM)] + [vspec] * 7,
        out_specs=[vspec] * 3,
        interpret=interpret,
    )(dtf.reshape(1, 1), ar.reshape(shp), ai.reshape(shp), wr.reshape(shp),
      wi.reshape(shp), th.reshape(shp), lr0.reshape(shp), li0.reshape(shp))
    return tuple(o.reshape(N) for o in outs)


def _build_collapse(interpret=False, num_cores=None, num_subcores=None):
    if num_cores is None:
        mesh = plsc.VectorSubcoreMesh(core_axis_name="c", subcore_axis_name="s")
    else:
        mesh = plsc.VectorSubcoreMesh(core_axis_name="c", subcore_axis_name="s",
                                      num_cores=num_cores,
                                      num_subcores=num_subcores)
    nc, ns = mesh.num_cores, mesh.num_subcores
    nw = nc * ns
    rpt = N // nw
    half = N // nc
    hchunks = half // LANES

    @functools.partial(
        pl.kernel,
        out_type=(jax.ShapeDtypeStruct((1, N), jnp.float32),
                  jax.ShapeDtypeStruct((1, N), jnp.float32)),
        mesh=mesh,
        interpret=interpret,
        compiler_params=pltpu.CompilerParams(needs_layout_passes=False),
        scratch_types=[
            pltpu.VMEM((N,), jnp.float32),
            pltpu.VMEM((N,), jnp.float32),
            pltpu.VMEM((half,), jnp.float32),
            pltpu.VMEM((half,), jnp.int32),
            pltpu.VMEM((LANES,), jnp.int32),
            pltpu.VMEM((LANES,), jnp.int32),
            pltpu.VMEM((BATCH, N), jnp.float32),
            pltpu.VMEM((BATCH, N), jnp.float32),
            pltpu.VMEM((rpt,), jnp.float32),
            pltpu.VMEM((rpt,), jnp.float32),
            pltpu.VMEM((LANES,), jnp.float32),
            pltpu.VMEM((LANES,), jnp.float32),
            pltpu.VMEM((half,), jnp.float32),
            pltpu.VMEM((half,), jnp.float32),
            pltpu.VMEM_SHARED((half,), jnp.int32),
            pltpu.VMEM_SHARED((LANES,), jnp.int32),
            pltpu.VMEM_SHARED((LANES,), jnp.int32),
            pltpu.VMEM_SHARED((half,), jnp.float32),
            pltpu.VMEM_SHARED((half,), jnp.float32),
            pltpu.SemaphoreType.DMA,
            pltpu.SemaphoreType.DMA,
        ],
    )
    def collapse(wr_hbm, wi_hbm, lr_hbm, li_hbm, amt_hbm,
                 outr_hbm, outi_hbm,
                 lr_v, li_v, amt_v, glist_v, bnd_v, tot_v, rowr_v, rowi_v,
                 outr_v, outi_v, fr8_v, fi8_v, resr_v, resi_v,
                 glist_s, bnd_s, tot_s, resr_s, resi_s, semr, semi):
        sc = lax.axis_index("c")
        tid = lax.axis_index("s")
        sc_base = sc * half
        row0 = sc_base + tid * rpt

        iota = lax.broadcasted_iota(jnp.int32, (LANES,), 0)
        lane0 = iota == 0

        pltpu.async_copy(lr_hbm, lr_v, semr)
        pltpu.async_copy(li_hbm, li_v, semr)
        pltpu.async_copy(lr_hbm.at[pl.ds(row0, rpt)], outr_v, semi)
        pltpu.async_copy(li_hbm.at[pl.ds(row0, rpt)], outi_v, semi)

        @pl.when(tid == 0)
        def _():
            pltpu.sync_copy(amt_hbm.at[pl.ds(sc_base, half)], amt_v)
            base_v = jnp.full((LANES,), sc_base, jnp.int32)
            for k in range(hchunks):
                glist_v[pl.ds(k * LANES, LANES)] = base_v
            cnt = jnp.int32(0)
            for k in range(hchunks):
                if k % (rpt // LANES) == 0:
                    plsc.store_scatter(
                        bnd_v, [jnp.full((LANES,), k // (rpt // LANES),
                                         jnp.int32)],
                        jnp.full((LANES,), cnt, jnp.int32), mask=lane0)
                m = amt_v[pl.ds(k * LANES, LANES)] >= 0.0
                mi = m.astype(jnp.int32)
                pos = cnt + jnp.cumsum(mi) - 1
                plsc.store_scatter(glist_v, [pos],
                                   base_v + (k * LANES) + iota, mask=m)
                cnt = cnt + jnp.sum(mi)
            tot_v[...] = jnp.full((LANES,), cnt, jnp.int32)
            pltpu.sync_copy(glist_v, glist_s)
            pltpu.sync_copy(bnd_v, bnd_s)
            pltpu.sync_copy(tot_v, tot_s)

        plsc.subcore_barrier()

        pltpu.sync_copy(glist_s, glist_v)
        pltpu.sync_copy(bnd_s, bnd_v)
        pltpu.sync_copy(tot_s, tot_v)
        total = tot_v[pl.ds(0, LANES)][0]

        share = ((total + (8 * ns - 1)) // (8 * ns)) * 8
        start = tid * share
        myn = jnp.maximum(0, jnp.minimum(total - start, share))
        nb = (myn + (BATCH - 1)) // BATCH
        zeros = jnp.zeros((LANES,), jnp.float32)

        pltpu.make_async_copy(lr_hbm.at[pl.ds(row0, rpt)], outr_v, semi).wait()
        pltpu.make_async_copy(li_hbm.at[pl.ds(row0, rpt)], outi_v, semi).wait()
        pltpu.make_async_copy(lr_hbm, lr_v, semr).wait()
        pltpu.make_async_copy(li_hbm, li_v, semr).wait()

        @pl.when(nb > 0)
        def _():
            pltpu.async_copy(wr_hbm.at[glist_v.at[pl.ds(start, BATCH)]],
                             rowr_v, semr)

        def batch_body(b, carry):
            sl = glist_v.at[pl.ds(start + b * BATCH, BATCH)]
            pltpu.make_async_copy(wr_hbm.at[sl], rowr_v, semr).wait()
            pltpu.async_copy(wi_hbm.at[sl], rowi_v, semi)

            init = (tuple([zeros] * BATCH), tuple([zeros] * BATCH))

            @plsc.parallel_loop(0, NCHUNK // 2, unroll=1, carry=init)
            def acc_a(c, accs):
                fr_t, fi_t = accs
                base = c * (2 * LANES)
                nfr, nfi = list(fr_t), list(fi_t)
                for h in range(2):
                    lr_c = lr_v[pl.ds(base + h * LANES, LANES)]
                    li_c = li_v[pl.ds(base + h * LANES, LANES)]
                    for r in range(BATCH):
                        w_r = rowr_v[r, pl.ds(base + h * LANES, LANES)]
                        nfr[r] = nfr[r] + w_r * lr_c
                        nfi[r] = nfi[r] + w_r * li_c
                return (tuple(nfr), tuple(nfi))

            fr_t, fi_t = acc_a

            pltpu.make_async_copy(wi_hbm.at[sl], rowi_v, semi).wait()

            @pl.when(b + 1 < nb)
            def _():
                pltpu.async_copy(
                    wr_hbm.at[glist_v.at[pl.ds(start + (b + 1) * BATCH,
                                               BATCH)]],
                    rowr_v, semr)

            @plsc.parallel_loop(0, NCHUNK // 2, unroll=1,
                                carry=(fr_t, fi_t))
            def acc_b(c, accs):
                fr_t, fi_t = accs
                base = c * (2 * LANES)
                nfr, nfi = list(fr_t), list(fi_t)
                for h in range(2):
                    lr_c = lr_v[pl.ds(base + h * LANES, LANES)]
                    li_c = li_v[pl.ds(base + h * LANES, LANES)]
                    for r in range(BATCH):
                        w_i = rowi_v[r, pl.ds(base + h * LANES, LANES)]
                        nfr[r] = nfr[r] - w_i * li_c
                        nfi[r] = nfi[r] + w_i * lr_c
                return (tuple(nfr), tuple(nfi))

            fr_t, fi_t = acc_b

            for r in range(BATCH):
                rl = jnp.full((LANES,), r, jnp.int32)
                plsc.store_scatter(fr8_v, [rl],
                                   jnp.full((LANES,), jnp.sum(fr_t[r])),
                                   mask=lane0)
                plsc.store_scatter(fi8_v, [rl],
                                   jnp.full((LANES,), jnp.sum(fi_t[r])),
                                   mask=lane0)
            pltpu.sync_copy(fr8_v.at[pl.ds(0, BATCH)],
                            resr_s.at[pl.ds(start + b * BATCH, BATCH)])
            pltpu.sync_copy(fi8_v.at[pl.ds(0, BATCH)],
                            resi_s.at[pl.ds(start + b * BATCH, BATCH)])
            return carry

        lax.fori_loop(0, nb, batch_body, jnp.int32(0))

        plsc.subcore_barrier()

        pltpu.sync_copy(resr_s, resr_v)
        pltpu.sync_copy(resi_s, resi_v)
        tid_v = jnp.full((LANES,), tid, jnp.int32)
        lo = plsc.load_gather(bnd_v, [tid_v])[0]
        hi_next = plsc.load_gather(bnd_v, [jnp.minimum(tid_v + 1, ns - 1)])[0]
        hi = jnp.where(tid == ns - 1, total, hi_next)
        row0_v = jnp.full((LANES,), row0, jnp.int32)

        def owner_body(s, carry):
            s_v = jnp.full((LANES,), s, jnp.int32)
            g_v = plsc.load_gather(glist_v, [s_v])
            l_v = g_v - row0_v
            frv = plsc.load_gather(resr_v, [s_v])
            fiv = plsc.load_gather(resi_v, [s_v])
            lrv = plsc.load_gather(lr_v, [g_v])
            liv = plsc.load_gather(li_v, [g_v])
            plsc.store_scatter(outr_v, [l_v],
                               lrv * jnp.maximum(frv, 0.0), mask=lane0)
            plsc.store_scatter(outi_v, [l_v], liv * fiv, mask=lane0)
            return carry

        lax.fori_loop(lo, hi, owner_body, jnp.int32(0))

        pltpu.sync_copy(outr_v, outr_hbm.at[0, pl.ds(row0, rpt)])
        pltpu.sync_copy(outi_v, outi_hbm.at[0, pl.ds(row0, rpt)])

    return collapse


def kernel(t_span, dt, A_real, A_imag, w_acc_real, w_acc_imag, theta,
           W_filter_real, W_filter_imag, L_real_init, L_imag_init):
    num_steps = t_span.shape[0] - 1
    dtf = jnp.asarray(dt, jnp.float32)
    collapse = _build_collapse()
    Lr, Li = L_real_init, L_imag_init
    reals, imags = [], []
    for _ in range(num_steps):
        lr1, li1, amt = _prep(dtf, A_real, A_imag, w_acc_real, w_acc_imag,
                              theta, Lr, Li)
        o_r, o_i = collapse(W_filter_real, W_filter_imag, lr1, li1, amt)
        Lr, Li = o_r[0], o_i[0]
        reals.append(o_r)
        imags.append(o_i)
    if num_steps == 1:
        return reals[0], imags[0]
    return jnp.concatenate(reals), jnp.concatenate(imags)


# speedup vs baseline: 1.3465x; 1.0565x over previous
import functools

import jax
import jax.numpy as jnp
from jax import lax
from jax.experimental import pallas as pl
from jax.experimental.pallas import tpu as pltpu
from jax.experimental.pallas import tpu_sc as plsc

N = 4096
LANES = 16
BATCH = 8
NCHUNK = N // LANES


def _prep_body(dt_ref, ar, ai, wr, wi, th, lr0, li0, olr, oli, oamt):
    dtf = dt_ref[0, 0]
    er = jnp.exp(ar[...] * dtf)
    exp_r = er * jnp.cos(ai[...] * dtf)
    exp_i = er * jnp.sin(ai[...] * dtf)
    lr = lr0[...] * exp_r - li0[...] * exp_i
    li = lr0[...] * exp_i + li0[...] * exp_r
    olr[...] = lr
    oli[...] = li
    oamt[...] = wr[...] * lr + wi[...] * li - th[...]


def _prep(dtf, ar, ai, wr, wi, th, lr0, li0):
    shp = (N // 128, 128)
    vspec = pl.BlockSpec(memory_space=pltpu.VMEM)
    outs = pl.pallas_call(
        _prep_body,
        out_shape=[jax.ShapeDtypeStruct(shp, jnp.float32)] * 3,
        in_specs=[pl.BlockSpec(memory_space=pltpu.SMEM)] + [vspec] * 7,
        out_specs=[vspec] * 3,
    )(dtf.reshape(1, 1), ar.reshape(shp), ai.reshape(shp), wr.reshape(shp),
      wi.reshape(shp), th.reshape(shp), lr0.reshape(shp), li0.reshape(shp))
    return tuple(o.reshape(N) for o in outs)


def _build_collapse():
    mesh = plsc.VectorSubcoreMesh(core_axis_name="c", subcore_axis_name="s")
    nc, ns = mesh.num_cores, mesh.num_subcores
    nw = nc * ns
    rpt = N // nw
    half = N // nc
    hchunks = half // LANES

    @functools.partial(
        pl.kernel,
        out_type=(jax.ShapeDtypeStruct((1, N), jnp.float32),
                  jax.ShapeDtypeStruct((1, N), jnp.float32)),
        mesh=mesh,
        compiler_params=pltpu.CompilerParams(needs_layout_passes=False),
        scratch_types=[
            pltpu.VMEM((N,), jnp.float32),
            pltpu.VMEM((N,), jnp.float32),
            pltpu.VMEM((half,), jnp.float32),
            pltpu.VMEM((half,), jnp.int32),
            pltpu.VMEM((LANES,), jnp.int32),
            pltpu.VMEM((LANES,), jnp.int32),
            pltpu.VMEM((BATCH, N), jnp.float32),
            pltpu.VMEM((BATCH, N), jnp.float32),
            pltpu.VMEM((rpt,), jnp.float32),
            pltpu.VMEM((rpt,), jnp.float32),
            pltpu.VMEM((LANES,), jnp.float32),
            pltpu.VMEM((LANES,), jnp.float32),
            pltpu.VMEM((half,), jnp.float32),
            pltpu.VMEM((half,), jnp.float32),
            pltpu.VMEM_SHARED((half,), jnp.int32),
            pltpu.VMEM_SHARED((LANES,), jnp.int32),
            pltpu.VMEM_SHARED((LANES,), jnp.int32),
            pltpu.VMEM_SHARED((half,), jnp.float32),
            pltpu.VMEM_SHARED((half,), jnp.float32),
            pltpu.SemaphoreType.DMA,
            pltpu.SemaphoreType.DMA,
        ],
    )
    def collapse(wr_hbm, wi_hbm, lr_hbm, li_hbm, amt_hbm,
                 outr_hbm, outi_hbm,
                 lr_v, li_v, amt_v, glist_v, bnd_v, tot_v, rowr_v, rowi_v,
                 outr_v, outi_v, fr8_v, fi8_v, resr_v, resi_v,
                 glist_s, bnd_s, tot_s, resr_s, resi_s, semr, semi):
        sc = lax.axis_index("c")
        tid = lax.axis_index("s")
        sc_base = sc * half
        row0 = sc_base + tid * rpt

        iota = lax.broadcasted_iota(jnp.int32, (LANES,), 0)
        lane0 = iota == 0

        pltpu.async_copy(lr_hbm, lr_v, semr)
        pltpu.async_copy(li_hbm, li_v, semr)
        pltpu.async_copy(lr_hbm.at[pl.ds(row0, rpt)], outr_v, semi)
        pltpu.async_copy(li_hbm.at[pl.ds(row0, rpt)], outi_v, semi)

        @pl.when(tid == 0)
        def _():
            pltpu.sync_copy(amt_hbm.at[pl.ds(sc_base, half)], amt_v)
            base_v = jnp.full((LANES,), sc_base, jnp.int32)
            for k in range(hchunks):
                glist_v[pl.ds(k * LANES, LANES)] = base_v
            cnt = jnp.int32(0)
            for k in range(hchunks):
                if k % (rpt // LANES) == 0:
                    plsc.store_scatter(
                        bnd_v, [jnp.full((LANES,), k // (rpt // LANES),
                                         jnp.int32)],
                        jnp.full((LANES,), cnt, jnp.int32), mask=lane0)
                m = amt_v[pl.ds(k * LANES, LANES)] >= 0.0
                mi = m.astype(jnp.int32)
                pos = cnt + jnp.cumsum(mi) - 1
                plsc.store_scatter(glist_v, [pos],
                                   base_v + (k * LANES) + iota, mask=m)
                cnt = cnt + jnp.sum(mi)
            tot_v[...] = jnp.full((LANES,), cnt, jnp.int32)
            pltpu.sync_copy(glist_v, glist_s)
            pltpu.sync_copy(bnd_v, bnd_s)
            pltpu.sync_copy(tot_v, tot_s)

        plsc.subcore_barrier()

        pltpu.sync_copy(glist_s, glist_v)
        pltpu.sync_copy(bnd_s, bnd_v)
        pltpu.sync_copy(tot_s, tot_v)
        total = tot_v[pl.ds(0, LANES)][0]

        share = ((total + (8 * ns - 1)) // (8 * ns)) * 8
        start = tid * share
        myn = jnp.maximum(0, jnp.minimum(total - start, share))
        nb = (myn + (BATCH - 1)) // BATCH
        zeros = jnp.zeros((LANES,), jnp.float32)

        pltpu.make_async_copy(lr_hbm.at[pl.ds(row0, rpt)], outr_v, semi).wait()
        pltpu.make_async_copy(li_hbm.at[pl.ds(row0, rpt)], outi_v, semi).wait()
        pltpu.make_async_copy(lr_hbm, lr_v, semr).wait()
        pltpu.make_async_copy(li_hbm, li_v, semr).wait()

        @pl.when(nb > 0)
        def _():
            pltpu.async_copy(wr_hbm.at[glist_v.at[pl.ds(start, BATCH)]],
                             rowr_v, semr)

        def batch_body(b, carry):
            sl = glist_v.at[pl.ds(start + b * BATCH, BATCH)]
            pltpu.make_async_copy(wr_hbm.at[sl], rowr_v, semr).wait()
            pltpu.async_copy(wi_hbm.at[sl], rowi_v, semi)

            init = (tuple([zeros] * BATCH), tuple([zeros] * BATCH))

            @plsc.parallel_loop(0, NCHUNK // 2, unroll=1, carry=init)
            def acc_a(c, accs):
                fr_t, fi_t = accs
                base = c * (2 * LANES)
                nfr, nfi = list(fr_t), list(fi_t)
                for h in range(2):
                    lr_c = lr_v[pl.ds(base + h * LANES, LANES)]
                    li_c = li_v[pl.ds(base + h * LANES, LANES)]
                    for r in range(BATCH):
                        w_r = rowr_v[r, pl.ds(base + h * LANES, LANES)]
                        nfr[r] = nfr[r] + w_r * lr_c
                        nfi[r] = nfi[r] + w_r * li_c
                return (tuple(nfr), tuple(nfi))

            fr_t, fi_t = acc_a

            pltpu.make_async_copy(wi_hbm.at[sl], rowi_v, semi).wait()

            @pl.when(b + 1 < nb)
            def _():
                pltpu.async_copy(
                    wr_hbm.at[glist_v.at[pl.ds(start + (b + 1) * BATCH,
                                               BATCH)]],
                    rowr_v, semr)

            @plsc.parallel_loop(0, NCHUNK // 2, unroll=1,
                                carry=(fr_t, fi_t))
            def acc_b(c, accs):
                fr_t, fi_t = accs
                base = c * (2 * LANES)
                nfr, nfi = list(fr_t), list(fi_t)
                for h in range(2):
                    lr_c = lr_v[pl.ds(base + h * LANES, LANES)]
                    li_c = li_v[pl.ds(base + h * LANES, LANES)]
                    for r in range(BATCH):
                        w_i = rowi_v[r, pl.ds(base + h * LANES, LANES)]
                        nfr[r] = nfr[r] - w_i * li_c
                        nfi[r] = nfi[r] + w_i * lr_c
                return (tuple(nfr), tuple(nfi))

            fr_t, fi_t = acc_b

            for r in range(BATCH):
                rl = jnp.full((LANES,), r, jnp.int32)
                plsc.store_scatter(fr8_v, [rl],
                                   jnp.full((LANES,), jnp.sum(fr_t[r])),
                                   mask=lane0)
                plsc.store_scatter(fi8_v, [rl],
                                   jnp.full((LANES,), jnp.sum(fi_t[r])),
                                   mask=lane0)
            pltpu.sync_copy(fr8_v.at[pl.ds(0, BATCH)],
                            resr_s.at[pl.ds(start + b * BATCH, BATCH)])
            pltpu.sync_copy(fi8_v.at[pl.ds(0, BATCH)],
                            resi_s.at[pl.ds(start + b * BATCH, BATCH)])
            return carry

        lax.fori_loop(0, nb, batch_body, jnp.int32(0))

        plsc.subcore_barrier()

        pltpu.sync_copy(resr_s, resr_v)
        pltpu.sync_copy(resi_s, resi_v)
        tid_v = jnp.full((LANES,), tid, jnp.int32)
        lo = plsc.load_gather(bnd_v, [tid_v])[0]
        hi_next = plsc.load_gather(bnd_v, [jnp.minimum(tid_v + 1, ns - 1)])[0]
        hi = jnp.where(tid == ns - 1, total, hi_next)
        row0_v = jnp.full((LANES,), row0, jnp.int32)

        def owner_body(s, carry):
            s_v = jnp.full((LANES,), s, jnp.int32)
            g_v = plsc.load_gather(glist_v, [s_v])
            l_v = g_v - row0_v
            frv = plsc.load_gather(resr_v, [s_v])
            fiv = plsc.load_gather(resi_v, [s_v])
            lrv = plsc.load_gather(lr_v, [g_v])
            liv = plsc.load_gather(li_v, [g_v])
            plsc.store_scatter(outr_v, [l_v],
                               lrv * jnp.maximum(frv, 0.0), mask=lane0)
            plsc.store_scatter(outi_v, [l_v], liv * fiv, mask=lane0)
            return carry

        lax.fori_loop(lo, hi, owner_body, jnp.int32(0))

        pltpu.sync_copy(outr_v, outr_hbm.at[0, pl.ds(row0, rpt)])
        pltpu.sync_copy(outi_v, outi_hbm.at[0, pl.ds(row0, rpt)])

    return collapse


def kernel(t_span, dt, A_real, A_imag, w_acc_real, w_acc_imag, theta,
           W_filter_real, W_filter_imag, L_real_init, L_imag_init):
    num_steps = t_span.shape[0] - 1
    dtf = jnp.asarray(dt, jnp.float32)
    collapse = _build_collapse()
    Lr, Li = L_real_init, L_imag_init
    reals, imags = [], []
    for _ in range(num_steps):
        lr1, li1, amt = _prep(dtf, A_real, A_imag, w_acc_real, w_acc_imag,
                              theta, Lr, Li)
        o_r, o_i = collapse(W_filter_real, W_filter_imag, lr1, li1, amt)
        Lr, Li = o_r[0], o_i[0]
        reals.append(o_r)
        imags.append(o_i)
    if num_steps == 1:
        return reals[0], imags[0]
    return jnp.concatenate(reals), jnp.concatenate(imags)
